# baseline (speedup 1.0000x reference)
import dataclasses
import functools

import jax
import jax.numpy as jnp
from jax import lax
from jax.experimental import pallas as pl
from jax.experimental.pallas import tpu as pltpu
from jax.experimental.pallas import tpu_sc as plsc

_N = 10000
_C = 128
_NPAD = 10240
_EROWS = 2592
_CH = 8
_SLICE = _NPAD // 16

def _mesh():
    return plsc.VectorSubcoreMesh(core_axis_name="c", subcore_axis_name="s")


def _sc_params():
    cp = pltpu.CompilerParams()
    if "needs_layout_passes" in pltpu.CompilerParams.__dataclass_fields__:
        cp = dataclasses.replace(cp, needs_layout_passes=False)
    return cp


def _embed_body(x_ref, w_ref, av_ref, bv_ref, h_ref, as_ref, ad_ref):
    h = jnp.dot(x_ref[...], w_ref[...], preferred_element_type=jnp.float32)
    h_ref[...] = h
    as_ref[...] = jnp.sum(h * av_ref[...][None, :], axis=1, keepdims=True)
    ad_ref[...] = jnp.sum(h * bv_ref[...][None, :], axis=1, keepdims=True)


def _tc_embed(x, w, av, bv):
    return pl.pallas_call(
        _embed_body,
        out_shape=(
            jax.ShapeDtypeStruct((_NPAD, _C), jnp.float32),
            jax.ShapeDtypeStruct((_NPAD, 1), jnp.float32),
            jax.ShapeDtypeStruct((_NPAD, 1), jnp.float32),
        ),
    )(x, w, av, bv)


def _mid_body(p0_ref, p1_ref, s0_ref, s1_ref, b_ref, w_ref, av_ref, bv_ref,
              h_ref, as_ref, ad_ref):
    inv = 1.0 / (s0_ref[...] + s1_ref[...] + 1e-16)
    x2 = jnp.maximum((p0_ref[...] + p1_ref[...]) * inv + b_ref[...][None, :],
                     0.0)
    h = jnp.dot(x2, w_ref[...], preferred_element_type=jnp.float32)
    h_ref[...] = h
    as_ref[...] = jnp.sum(h * av_ref[...][None, :], axis=1, keepdims=True)
    ad_ref[...] = jnp.sum(h * bv_ref[...][None, :], axis=1, keepdims=True)


def _tc_mid(p0, p1, s0, s1, b, w, av, bv):
    return pl.pallas_call(
        _mid_body,
        out_shape=(
            jax.ShapeDtypeStruct((_NPAD, _C), jnp.float32),
            jax.ShapeDtypeStruct((_NPAD, 1), jnp.float32),
            jax.ShapeDtypeStruct((_NPAD, 1), jnp.float32),
        ),
    )(p0, p1, s0, s1, b, w, av, bv)


def _final_body(p0_ref, p1_ref, s0_ref, s1_ref, b_ref, o_ref):
    inv = 1.0 / (s0_ref[...] + s1_ref[...] + 1e-16)
    o_ref[...] = (p0_ref[...] + p1_ref[...]) * inv + b_ref[...][None, :]


def _tc_final(p0, p1, s0, s1, b):
    return pl.pallas_call(
        _final_body,
        out_shape=jax.ShapeDtypeStruct((_NPAD, _C), jnp.float32),
    )(p0, p1, s0, s1, b)


def _sc_layer(alpha_src, alpha_dst, h, src2, dst2):
    @functools.partial(
        pl.kernel,
        mesh=_mesh(),
        compiler_params=_sc_params(),
        out_type=(
            jax.ShapeDtypeStruct((2, _NPAD), jnp.float32),
            jax.ShapeDtypeStruct((2, _NPAD, _C), jnp.float32),
        ),
        scratch_types=[
            pltpu.VMEM((_NPAD,), jnp.float32),
            pltpu.VMEM((_NPAD,), jnp.float32),
            pltpu.VMEM((_CH, 128), jnp.int32),
            pltpu.VMEM((_CH, 128), jnp.int32),
            pltpu.VMEM((128,), jnp.float32),
            pltpu.VMEM((_SLICE,), jnp.float32),
            pltpu.VMEM((2, 64, _C), jnp.float32),
            pltpu.VMEM((64, _C), jnp.float32),
            pltpu.VMEM_SHARED((_NPAD,), jnp.float32),
            pltpu.VMEM_SHARED((_NPAD, _C), jnp.float32),
            pltpu.SemaphoreType.DMA,
            pltpu.SemaphoreType.DMA,
        ],
    )
    def k(as_hbm, ad_hbm, h_hbm, src_hbm, dst_hbm, sp_hbm, out_hbm,
          as_v, ad_v, src_v, dst_v, ex_v, zero_v, rows_v, frow_v, s_sh,
          acc_sh, gs0, gs1):
        cid = lax.axis_index("c")
        sid = lax.axis_index("s")
        wid = sid * 2 + cid

        pltpu.sync_copy(as_hbm, as_v)
        pltpu.sync_copy(ad_hbm, ad_v)

        def _mx(i, acc):
            return jnp.maximum(acc, as_v[pl.ds(i * 16, 16)])

        mx = lax.fori_loop(0, _NPAD // 16, _mx,
                           jnp.full((16,), -3.0e38, jnp.float32))
        asmax = jnp.max(mx)

        @pl.loop(0, 64)
        def _(rr):
            zrow = frow_v.at[rr]
            for c in range(8):
                zrow[pl.ds(c * 16, 16)] = jnp.zeros((16,), jnp.float32)

        @pl.loop(0, _SLICE // 16)
        def _(i):
            zero_v[pl.ds(i * 16, 16)] = jnp.zeros((16,), jnp.float32)

        for t in range(10):
            pltpu.sync_copy(frow_v,
                            acc_sh.at[pl.ds(sid * _SLICE + t * 64, 64)])
        pltpu.sync_copy(zero_v, s_sh.at[pl.ds(sid * _SLICE, _SLICE)])
        plsc.subcore_barrier()

        gs = (gs0, gs1)

        @pl.loop(wid * _CH, _EROWS, step=32 * _CH)
        def _(r0):
            pltpu.sync_copy(src_hbm.at[pl.ds(r0, _CH)], src_v)
            pltpu.sync_copy(dst_hbm.at[pl.ds(r0, _CH)], dst_v)
            gcp = [pltpu.make_async_copy(
                       h_hbm.at[src_v.at[jj // 2, pl.ds((jj % 2) * 64, 64)]],
                       rows_v.at[jj % 2], gs[jj % 2])
                   for jj in range(2 * _CH)]
            gcp[0].start()
            for jj in range(2 * _CH):
                j, hh = jj // 2, jj % 2
                srow = src_v.at[j]
                drow = dst_v.at[j]
                for c in range(4):
                    sl = pl.ds(hh * 64 + c * 16, 16)
                    av = plsc.load_gather(as_v, [srow[sl]])
                    bv = plsc.load_gather(ad_v, [drow[sl]])
                    z = av + bv
                    e = jnp.where(z > 0, z, 0.2 * z)
                    zu = asmax + bv
                    mub = jnp.where(zu > 0, zu, 0.2 * zu)
                    ex_v[sl] = jnp.exp(e - mub)
                if jj + 1 < 2 * _CH:
                    gcp[jj + 1].start()
                pltpu.sync_copy(ex_v.at[pl.ds(hh * 64, 64)],
                                s_sh.at[dst_v.at[j, pl.ds(hh * 64, 64)]],
                                add=True)
                gcp[jj].wait()
                rb = rows_v.at[jj % 2]

                @pl.loop(0, 4)
                def _(g):
                    a16 = ex_v[pl.ds(hh * 64 + g * 16, 16)]
                    for i in range(16):
                        ai = a16[i]
                        row = rb.at[g * 16 + i]
                        for c in range(8):
                            sl16 = pl.ds(c * 16, 16)
                            row[sl16] = row[sl16] * ai

                pltpu.sync_copy(rb,
                                acc_sh.at[dst_v.at[j, pl.ds(hh * 64, 64)]],
                                add=True)

        plsc.subcore_barrier()
        pltpu.sync_copy(s_sh.at[pl.ds(sid * _SLICE, _SLICE)],
                        sp_hbm.at[cid, pl.ds(sid * _SLICE, _SLICE)])
        pltpu.sync_copy(acc_sh.at[pl.ds(sid * _SLICE, _SLICE)],
                        out_hbm.at[cid, pl.ds(sid * _SLICE, _SLICE)])

    return k(alpha_src, alpha_dst, h, src2, dst2)


def _layer(hb, a_s, a_d, src2, dst2):
    sp, up = _sc_layer(a_s.reshape(-1), a_d.reshape(-1), hb, src2, dst2)
    s3 = sp.reshape(2, _NPAD, 1)
    return up, s3


@jax.jit
def kernel(x, edge_index, W1, a_src1, a_dst1, b1, W2, a_src2, a_dst2, b2):
    ei = edge_index.astype(jnp.int32)
    loops = jnp.arange(_NPAD, dtype=jnp.int32)
    npadding = _EROWS * 128 - ei.shape[1] - _NPAD
    padv = jnp.full((npadding,), _NPAD - 1, jnp.int32)
    src2 = jnp.concatenate([ei[0], loops, padv]).reshape(_EROWS, 128)
    dst2 = jnp.concatenate([ei[1], loops, padv]).reshape(_EROWS, 128)
    x_pad = jnp.pad(x, ((0, _NPAD - _N), (0, 0)))

    h1, as1, ad1 = _tc_embed(x_pad, W1, a_src1[0], a_dst1[0])
    u1, s1 = _layer(h1, as1, ad1, src2, dst2)
    h2, as2, ad2 = _tc_mid(u1[0], u1[1], s1[0], s1[1], b1,
                           W2, a_src2[0], a_dst2[0])
    u2, s2 = _layer(h2, as2, ad2, src2, dst2)
    out = _tc_final(u2[0], u2[1], s2[0], s2[1], b2)
    return out[:_N]


# speedup vs baseline: 1.0782x; 1.0782x over previous
import dataclasses
import functools

import jax
import jax.numpy as jnp
from jax import lax
from jax.experimental import pallas as pl
from jax.experimental.pallas import tpu as pltpu
from jax.experimental.pallas import tpu_sc as plsc

_N = 10000
_C = 128
_NPAD = 10240
_EROWS = 2592
_CH = 8
_SLICE = _NPAD // 16

def _mesh():
    return plsc.VectorSubcoreMesh(core_axis_name="c", subcore_axis_name="s")


def _sc_params():
    cp = pltpu.CompilerParams()
    if "needs_layout_passes" in pltpu.CompilerParams.__dataclass_fields__:
        cp = dataclasses.replace(cp, needs_layout_passes=False)
    return cp


def _embed_body(x_ref, w_ref, av_ref, bv_ref, h_ref, as_ref, ad_ref):
    h = jnp.dot(x_ref[...], w_ref[...], preferred_element_type=jnp.float32)
    h_ref[...] = h
    as_ref[...] = jnp.sum(h * av_ref[...][None, :], axis=1, keepdims=True)
    ad_ref[...] = jnp.sum(h * bv_ref[...][None, :], axis=1, keepdims=True)


def _tc_embed(x, w, av, bv):
    return pl.pallas_call(
        _embed_body,
        out_shape=(
            jax.ShapeDtypeStruct((_NPAD, _C), jnp.float32),
            jax.ShapeDtypeStruct((_NPAD, 1), jnp.float32),
            jax.ShapeDtypeStruct((_NPAD, 1), jnp.float32),
        ),
    )(x, w, av, bv)


def _mid_body(p0_ref, p1_ref, s0_ref, s1_ref, b_ref, w_ref, av_ref, bv_ref,
              h_ref, as_ref, ad_ref):
    inv = 1.0 / (s0_ref[...] + s1_ref[...] + 1e-16)
    x2 = jnp.maximum((p0_ref[...] + p1_ref[...]) * inv + b_ref[...][None, :],
                     0.0)
    h = jnp.dot(x2, w_ref[...], preferred_element_type=jnp.float32)
    h_ref[...] = h
    as_ref[...] = jnp.sum(h * av_ref[...][None, :], axis=1, keepdims=True)
    ad_ref[...] = jnp.sum(h * bv_ref[...][None, :], axis=1, keepdims=True)


def _tc_mid(p0, p1, s0, s1, b, w, av, bv):
    return pl.pallas_call(
        _mid_body,
        out_shape=(
            jax.ShapeDtypeStruct((_NPAD, _C), jnp.float32),
            jax.ShapeDtypeStruct((_NPAD, 1), jnp.float32),
            jax.ShapeDtypeStruct((_NPAD, 1), jnp.float32),
        ),
    )(p0, p1, s0, s1, b, w, av, bv)


def _final_body(p0_ref, p1_ref, s0_ref, s1_ref, b_ref, o_ref):
    inv = 1.0 / (s0_ref[...] + s1_ref[...] + 1e-16)
    o_ref[...] = (p0_ref[...] + p1_ref[...]) * inv + b_ref[...][None, :]


def _tc_final(p0, p1, s0, s1, b):
    return pl.pallas_call(
        _final_body,
        out_shape=jax.ShapeDtypeStruct((_NPAD, _C), jnp.float32),
    )(p0, p1, s0, s1, b)


def _sc_layer(alpha_src, alpha_dst, h, src2, dst2):
    @functools.partial(
        pl.kernel,
        mesh=_mesh(),
        compiler_params=_sc_params(),
        out_type=(
            jax.ShapeDtypeStruct((2, _NPAD), jnp.float32),
            jax.ShapeDtypeStruct((2, _NPAD, _C), jnp.float32),
        ),
        scratch_types=[
            pltpu.VMEM((_NPAD,), jnp.float32),
            pltpu.VMEM((_NPAD,), jnp.float32),
            pltpu.VMEM((_CH, 128), jnp.int32),
            pltpu.VMEM((_CH, 128), jnp.int32),
            pltpu.VMEM((_CH, 128), jnp.float32),
            pltpu.VMEM((3, 64, _C), jnp.float32),
            pltpu.VMEM_SHARED((_NPAD,), jnp.float32),
            pltpu.VMEM_SHARED((_NPAD, _C), jnp.float32),
            pltpu.SemaphoreType.DMA,
            pltpu.SemaphoreType.DMA,
            pltpu.SemaphoreType.DMA,
            pltpu.SemaphoreType.DMA,
            pltpu.SemaphoreType.DMA,
            pltpu.SemaphoreType.DMA,
            pltpu.SemaphoreType.DMA,
        ],
    )
    def k(as_hbm, ad_hbm, h_hbm, src_hbm, dst_hbm, sp_hbm, out_hbm,
          as_v, ad_v, src_v, dst_v, ex_v, rows_v, s_sh,
          acc_sh, gs0, gs1, gs2, rs0, rs1, rs2, es):
        cid = lax.axis_index("c")
        sid = lax.axis_index("s")
        wid = sid * 2 + cid

        pltpu.sync_copy(as_hbm, as_v)
        pltpu.sync_copy(ad_hbm, ad_v)

        def _mx(i, acc):
            return jnp.maximum(acc, as_v[pl.ds(i * 16, 16)])

        mx = lax.fori_loop(0, _NPAD // 16, _mx,
                           jnp.full((16,), -3.0e38, jnp.float32))
        asmax = jnp.max(mx)

        @pl.loop(0, 64)
        def _(rr):
            zrow = rows_v.at[0, rr]
            for c in range(8):
                zrow[pl.ds(c * 16, 16)] = jnp.zeros((16,), jnp.float32)

        for t in range(10):
            pltpu.sync_copy(rows_v.at[0],
                            acc_sh.at[pl.ds(sid * _SLICE + t * 64, 64)])
        for t in range(5):
            pltpu.sync_copy(rows_v.at[0, t],
                            s_sh.at[pl.ds(sid * _SLICE + t * 128, 128)])
        plsc.subcore_barrier()

        gs = (gs0, gs1, gs2)
        rs = (rs0, rs1, rs2)
        nh = 2 * _CH

        @pl.loop(wid * _CH, _EROWS, step=32 * _CH)
        def _(r0):
            pltpu.sync_copy(src_hbm.at[pl.ds(r0, _CH)], src_v)
            pltpu.sync_copy(dst_hbm.at[pl.ds(r0, _CH)], dst_v)
            def gstart(jj):
                return pltpu.async_copy(
                    h_hbm.at[src_v.at[jj // 2, pl.ds((jj % 2) * 64, 64)]],
                    rows_v.at[jj % 3], gs[jj % 3])

            gh = [None] * nh
            rh = [None] * nh
            eh = [None] * nh
            gh[0] = gstart(0)
            gh[1] = gstart(1)
            for jj in range(nh):
                j, hh = jj // 2, jj % 2
                srow = src_v.at[j]
                drow = dst_v.at[j]
                exr = ex_v.at[j]
                for c in range(4):
                    sl = pl.ds(hh * 64 + c * 16, 16)
                    av = plsc.load_gather(as_v, [srow[sl]])
                    bv = plsc.load_gather(ad_v, [drow[sl]])
                    z = av + bv
                    e = jnp.where(z > 0, z, 0.2 * z)
                    zu = asmax + bv
                    mub = jnp.where(zu > 0, zu, 0.2 * zu)
                    exr[sl] = jnp.exp(e - mub)
                eh[jj] = pltpu.async_copy(
                    ex_v.at[j, pl.ds(hh * 64, 64)],
                    s_sh.at[dst_v.at[j, pl.ds(hh * 64, 64)]],
                    es, add=True)
                if jj + 2 < nh:
                    if jj >= 1:
                        rh[jj - 1].wait()
                    gh[jj + 2] = gstart(jj + 2)
                gh[jj].wait()
                rb = rows_v.at[jj % 3]

                @pl.loop(0, 4)
                def _(g):
                    a16 = ex_v[j, pl.ds(hh * 64 + g * 16, 16)]
                    for i in range(16):
                        ai = a16[i]
                        row = rb.at[g * 16 + i]
                        for c in range(8):
                            sl16 = pl.ds(c * 16, 16)
                            row[sl16] = row[sl16] * ai

                rh[jj] = pltpu.async_copy(
                    rb,
                    acc_sh.at[dst_v.at[j, pl.ds(hh * 64, 64)]],
                    rs[jj % 3], add=True)
            for jj in range(nh - 3, nh):
                rh[jj].wait()
            for jj in range(nh):
                eh[jj].wait()

        plsc.subcore_barrier()
        pltpu.sync_copy(s_sh.at[pl.ds(sid * _SLICE, _SLICE)],
                        sp_hbm.at[cid, pl.ds(sid * _SLICE, _SLICE)])
        pltpu.sync_copy(acc_sh.at[pl.ds(sid * _SLICE, _SLICE)],
                        out_hbm.at[cid, pl.ds(sid * _SLICE, _SLICE)])

    return k(alpha_src, alpha_dst, h, src2, dst2)


def _layer(hb, a_s, a_d, src2, dst2):
    sp, up = _sc_layer(a_s.reshape(-1), a_d.reshape(-1), hb, src2, dst2)
    s3 = sp.reshape(2, _NPAD, 1)
    return up, s3


@jax.jit
def kernel(x, edge_index, W1, a_src1, a_dst1, b1, W2, a_src2, a_dst2, b2):
    ei = edge_index.astype(jnp.int32)
    loops = jnp.arange(_NPAD, dtype=jnp.int32)
    npadding = _EROWS * 128 - ei.shape[1] - _NPAD
    padv = jnp.full((npadding,), _NPAD - 1, jnp.int32)
    src2 = jnp.concatenate([ei[0], loops, padv]).reshape(_EROWS, 128)
    dst2 = jnp.concatenate([ei[1], loops, padv]).reshape(_EROWS, 128)
    x_pad = jnp.pad(x, ((0, _NPAD - _N), (0, 0)))

    h1, as1, ad1 = _tc_embed(x_pad, W1, a_src1[0], a_dst1[0])
    u1, s1 = _layer(h1, as1, ad1, src2, dst2)
    h2, as2, ad2 = _tc_mid(u1[0], u1[1], s1[0], s1[1], b1,
                           W2, a_src2[0], a_dst2[0])
    u2, s2 = _layer(h2, as2, ad2, src2, dst2)
    out = _tc_final(u2[0], u2[1], s2[0], s2[1], b2)
    return out[:_N]
